# baseline (speedup 1.0000x reference)
import jax
import jax.numpy as jnp
from jax import lax
from jax.experimental import pallas as pl
from jax.experimental.pallas import tpu as pltpu
from jax.experimental.pallas import tpu_sc as plsc

DIM = 64
BATCH = 16384
L = 16
NC = 2
NS = 16
NW = NC * NS
B_W = BATCH // NW
CHUNK = 128


def _tec_body(ent_hbm, rel_hbm, h_hbm, r_hbm, t_hbm, psum_hbm,
              hidx, ridx, tidx, hbuf, rbuf, tbuf, pbuf, sem):
    wid = lax.axis_index("s") * NC + lax.axis_index("c")
    base = wid * B_W

    pltpu.sync_copy(h_hbm.at[pl.ds(base, B_W)], hidx)
    pltpu.sync_copy(r_hbm.at[pl.ds(base, B_W)], ridx)
    pltpu.sync_copy(t_hbm.at[pl.ds(base, B_W)], tidx)

    copies = []
    for j in range(B_W // CHUNK):
        sl = pl.ds(j * CHUNK, CHUNK)
        copies.append(pltpu.async_copy(ent_hbm.at[hidx.at[sl]], hbuf.at[sl], sem))
        copies.append(pltpu.async_copy(rel_hbm.at[ridx.at[sl]], rbuf.at[sl], sem))
        copies.append(pltpu.async_copy(ent_hbm.at[tidx.at[sl]], tbuf.at[sl], sem))
    for c in copies:
        c.wait()

    def elem(i, carry):
        acc = jnp.zeros((L,), jnp.float32)
        for k in range(DIM // L):
            sl = pl.ds(k * L, L)
            diff = hbuf[i, sl] + rbuf[i, sl] - tbuf[i, sl]
            acc = acc + diff * diff
        pbuf[i, :] = acc
        return carry

    lax.fori_loop(0, B_W, elem, 0)

    pltpu.sync_copy(pbuf, psum_hbm.at[pl.ds(base, B_W)])


def _tc_norm_body(p_ref, o_ref):
    p = p_ref[...]
    lane_grp = lax.broadcasted_iota(jnp.int32, (128, 8), 0) // L
    out_grp = lax.broadcasted_iota(jnp.int32, (128, 8), 1)
    sel = (lane_grp == out_grp).astype(jnp.float32)
    o_ref[...] = jnp.sqrt(
        lax.dot_general(p, sel, (((1,), (0,)), ((), ())),
                        precision=lax.Precision.HIGHEST,
                        preferred_element_type=jnp.float32))


def kernel(ent_emb, rel_emb, h, r, t):
    h = h.astype(jnp.int32)
    r = r.astype(jnp.int32)
    t = t.astype(jnp.int32)
    mesh = plsc.VectorSubcoreMesh(core_axis_name="c", subcore_axis_name="s")
    gather_partials = pl.kernel(
        _tec_body,
        mesh=mesh,
        compiler_params=pltpu.CompilerParams(use_tc_tiling_on_sc=False),
        out_type=jax.ShapeDtypeStruct((BATCH, L), jnp.float32),
        scratch_types=[
            pltpu.VMEM((B_W,), jnp.int32),
            pltpu.VMEM((B_W,), jnp.int32),
            pltpu.VMEM((B_W,), jnp.int32),
            pltpu.VMEM((B_W, DIM), jnp.float32),
            pltpu.VMEM((B_W, DIM), jnp.float32),
            pltpu.VMEM((B_W, DIM), jnp.float32),
            pltpu.VMEM((B_W, L), jnp.float32),
            pltpu.SemaphoreType.DMA,
        ],
    )
    psums = gather_partials(ent_emb, rel_emb, h, r, t)
    norms = pl.pallas_call(
        _tc_norm_body,
        out_shape=jax.ShapeDtypeStruct((BATCH // 8, 8), jnp.float32),
    )(psums.reshape(BATCH // 8, 8 * L))
    return norms.reshape(BATCH)
